# Initial kernel scaffold; baseline (speedup 1.0000x reference)
import jax
import jax.numpy as jnp
from jax.experimental import pallas as pl


def kernel(logits, labels):
    raise NotImplementedError("write your pallas kernel here")


# speedup vs baseline: 38.5467x; 38.5467x over previous
import functools

import jax
import jax.numpy as jnp
from jax import lax
from jax.experimental import pallas as pl
from jax.experimental.pallas import tpu as pltpu
from jax.experimental.pallas import tpu_sc as plsc

NB = 1024
NC, NS, L = 2, 16, 16
NW = NC * NS
P = 1024
CPAD = 32


def _sc_hist_fn(B, C, N):
    SH = NW // B
    NP = N // SH
    NCH = NP // P
    CH2 = C * 2 * NB
    NBf = float(NB)

    mesh = plsc.VectorSubcoreMesh(
        core_axis_name="c", subcore_axis_name="s",
        num_cores=NC, num_subcores=NS)

    @functools.partial(
        pl.kernel,
        out_type=jax.ShapeDtypeStruct((SH, B, CH2), jnp.int32),
        mesh=mesh,
        compiler_params=pltpu.CompilerParams(needs_layout_passes=False),
        scratch_types=[
            pltpu.VMEM((C, P), jnp.float32),
            pltpu.VMEM((P,), jnp.int32),
            pltpu.VMEM((CPAD * P,), jnp.int32),
            pltpu.VMEM((CH2,), jnp.int32),
        ],
    )
    def k(logits_hbm, labels_hbm, out_hbm, lbuf, labbuf, gbuf, hist):
        wid = lax.axis_index("s") * NC + lax.axis_index("c")
        b = wid // SH
        sh = wid % SH

        zeros16 = jnp.zeros((L,), jnp.int32)
        ones16 = jnp.ones((L,), jnp.int32)
        lane = lax.iota(jnp.int32, L)
        lane_p = lane * P
        mask_hi = lane < (C - L)

        def zbody(i, carry):
            hist[pl.ds(i * L, L)] = zeros16
            return carry
        lax.fori_loop(0, CH2 // L, zbody, 0)

        def chunk_body(g, carry):
            base = sh * NP + g * P
            pltpu.sync_copy(logits_hbm.at[b, :, pl.ds(base, P)], lbuf)
            pltpu.sync_copy(labels_hbm.at[b, pl.ds(base, P)], labbuf)

            def vec_body(v, vcarry):
                off = v * L
                lab = labbuf[pl.ds(off, L)]
                es = []
                s = None
                for c in range(C):
                    e = jnp.exp(lbuf[c, pl.ds(off, L)])
                    es.append(e)
                    s = e if s is None else s + e
                rsN = NBf / s
                for c in range(C):
                    pe = es[c] * rsN
                    fg = lab == c
                    errN = jnp.where(fg, NBf - pe, pe)
                    binv = jnp.minimum(errN.astype(jnp.int32), NB - 1)
                    gb = binv + jnp.where(fg, c * 2 * NB + NB, c * 2 * NB)
                    gbuf[pl.ds(c * P + off, L)] = gb
                return vcarry
            lax.fori_loop(0, P // L, vec_body, 0)

            def scat_body(q, scarry):
                idx0 = lane_p + q
                g0 = plsc.load_gather(gbuf, [idx0])
                plsc.addupdate_scatter(hist, [g0], ones16)
                idx1 = idx0 + L * P
                g1 = plsc.load_gather(gbuf, [idx1])
                plsc.addupdate_scatter(hist, [g1], ones16, mask=mask_hi)
                return scarry
            lax.fori_loop(0, P, scat_body, 0)
            return carry
        lax.fori_loop(0, NCH, chunk_body, 0)

        pltpu.sync_copy(hist, out_hbm.at[sh, b])

    return k


def _finalize_fn(B, C, N):
    SH = NW // B
    BC = B * C

    def body(h_ref, o_ref):
        h = h_ref[...].astype(jnp.float32)
        hs = jnp.sum(h, axis=0)
        c0 = hs[:, :NB]
        c1 = hs[:, NB:]
        i_r = lax.broadcasted_iota(jnp.int32, (NB, NB), 0)
        i_c = lax.broadcasted_iota(jnp.int32, (NB, NB), 1)
        m = (i_r <= i_c).astype(jnp.float32)
        a0 = jnp.dot(c0, m, preferred_element_type=jnp.float32)
        a1 = jnp.dot(c1, m, preferred_element_type=jnp.float32)
        b0 = a0 - c0
        b1 = a1 - c1
        tot = jnp.float32(N)
        d_a = jnp.maximum(tot - a0, 0.5)
        d_b = jnp.maximum(tot - b0, 0.5)
        jd = a1 / d_a - b1 / d_b
        ehat = (lax.broadcasted_iota(jnp.int32, (1, NB), 1).astype(jnp.float32)
                + 0.5) / NB
        losses = jnp.sum(ehat * jd, axis=1)
        gcnt = jnp.sum(c1, axis=1)
        pres = (gcnt > 0).astype(jnp.float32)
        total = jnp.sum(losses * pres)
        cnt = jnp.sum(pres)
        val = jnp.where(cnt > 0, total / cnt, jnp.float32(0.0))
        o_ref[...] = jnp.broadcast_to(val, (1, 1))

    return pl.pallas_call(
        body,
        out_shape=jax.ShapeDtypeStruct((1, 1), jnp.float32),
    )


def kernel(logits, labels):
    B, C, N = logits.shape
    hist = _sc_hist_fn(B, C, N)(logits, labels.astype(jnp.int32))
    SH = NW // B
    h3 = hist.reshape(SH, B * C, 2 * NB)
    out = _finalize_fn(B, C, N)(h3)
    return out.reshape(())
